# Initial kernel scaffold; baseline (speedup 1.0000x reference)
import jax
import jax.numpy as jnp
from jax.experimental import pallas as pl


def kernel(tokens, embedding):
    raise NotImplementedError("write your pallas kernel here")


# speedup vs baseline: 1.0153x; 1.0153x over previous
import math

import jax
import jax.numpy as jnp
from jax import lax
from jax.experimental import pallas as pl
from jax.experimental.pallas import tpu as pltpu
from jax.experimental.pallas import tpu_sc as plsc

_NC = 2
_NS = 16
_NW = _NC * _NS

_BATCH = 16384
_SEQ = 50
_EMB = 32
_TOTAL = _BATCH * _SEQ
_SCALE = math.sqrt(float(_EMB))

_IDX_W = 128
_CH_ROWS = 8
_CH = _CH_ROWS * _IDX_W
_ROWS_PER_W = _TOTAL // _NW // _IDX_W
_NCHUNK = _ROWS_PER_W // _CH_ROWS


def _emb_body(tok_hbm, tab_hbm, out_hbm, idx_v, rows_v, sem):
    wid = lax.axis_index("s") * _NC + lax.axis_index("c")
    base_row = wid * _ROWS_PER_W

    def chunk(c, carry):
        row = base_row + c * _CH_ROWS
        pltpu.sync_copy(tok_hbm.at[pl.ds(row, _CH_ROWS)], idx_v)
        cps = [
            pltpu.async_copy(
                tab_hbm.at[idx_v.at[j]],
                rows_v.at[pl.ds(j * _IDX_W, _IDX_W)],
                sem,
            )
            for j in range(_CH_ROWS)
        ]
        for cp in cps:
            cp.wait()

        def scale(i, carry2):
            for u in range(4):
                r = i * 4 + u
                rows_v[r, pl.ds(0, 16)] = rows_v[r, pl.ds(0, 16)] * _SCALE
                rows_v[r, pl.ds(16, 16)] = rows_v[r, pl.ds(16, 16)] * _SCALE
            return carry2

        lax.fori_loop(0, _CH // 4, scale, 0)
        pltpu.sync_copy(rows_v, out_hbm.at[pl.ds(row * _IDX_W, _CH)])
        return carry

    lax.fori_loop(0, _NCHUNK, chunk, 0)


_mesh = plsc.VectorSubcoreMesh(
    core_axis_name="c", subcore_axis_name="s", num_cores=_NC, num_subcores=_NS
)

_emb_call = pl.kernel(
    _emb_body,
    out_type=jax.ShapeDtypeStruct((_TOTAL, _EMB), jnp.float32),
    mesh=_mesh,
    scratch_types=[
        pltpu.VMEM((_CH_ROWS, _IDX_W), jnp.int32),
        pltpu.VMEM((_CH, _EMB), jnp.float32),
        pltpu.SemaphoreType.DMA,
    ],
    compiler_params=pltpu.CompilerParams(use_tc_tiling_on_sc=False),
)


@jax.jit
def kernel(tokens, embedding):
    tok = tokens.reshape(_TOTAL // _IDX_W, _IDX_W).astype(jnp.int32)
    out = _emb_call(tok, embedding)
    return out.reshape(_BATCH, _SEQ, _EMB)
